# Initial kernel scaffold; baseline (speedup 1.0000x reference)
import jax
import jax.numpy as jnp
from jax.experimental import pallas as pl


def kernel(mem, idx, val, W1, b1, W2, b2):
    raise NotImplementedError("write your pallas kernel here")


# speedup vs baseline: 1.2086x; 1.2086x over previous
import functools

import jax
import jax.numpy as jnp
from jax import lax
from jax.experimental import pallas as pl
from jax.experimental.pallas import tpu as pltpu
from jax.experimental.pallas import tpu_sc as plsc

M = 100000
D = 128
H = 256
B = 16384

NC = 2
NS = 16
LANES = 16

PPC = 4
LMAX = 12504
DUMP = LMAX
NDUMP = 64
BPT = B // NS
NG = BPT // LANES
IB = 128
NB = BPT // IB
LROWS = NB + 2

BM = 2048


def _mlp_body(val_ref, w1_ref, b1_ref, w2_ref, b2_ref, out_ref):
    h = jnp.dot(val_ref[...], w1_ref[...], preferred_element_type=jnp.float32)
    h = jnp.maximum(h + b1_ref[...], 0.0)
    out_ref[...] = (
        jnp.dot(h, w2_ref[...], preferred_element_type=jnp.float32) + b2_ref[...]
    )


_mlp = pl.pallas_call(
    _mlp_body,
    grid=(B // BM,),
    in_specs=[
        pl.BlockSpec((BM, D), lambda i: (i, 0)),
        pl.BlockSpec((D, H), lambda i: (0, 0)),
        pl.BlockSpec((1, H), lambda i: (0, 0)),
        pl.BlockSpec((H, D), lambda i: (0, 0)),
        pl.BlockSpec((1, D), lambda i: (0, 0)),
    ],
    out_specs=pl.BlockSpec((BM, D), lambda i: (i, 0)),
    out_shape=jax.ShapeDtypeStruct((B, D), jnp.float32),
)


def _first(vec):
    e0 = jnp.sum(jnp.where(lax.iota(jnp.int32, LANES) == 0, vec, 0))
    return jnp.broadcast_to(e0, (LANES,))


def _sc_body(idx_hbm, upd_hbm, mem_hbm, out_hbm,
             idx_v, pos_s, org_s, adj_s, adjc_s, rowbuf, acc_sh):
    c = lax.axis_index("c")
    s = lax.axis_index("s")
    row0 = s * BPT
    pltpu.sync_copy(idx_hbm.at[pl.ds(row0, BPT)], idx_v)
    iota = lax.iota(jnp.int32, LANES)

    def put(dst, pos, org, adj, adjc, m):
        r = lax.shift_right_logical(dst, 7)
        q = dst & (IB - 1)
        plsc.store_scatter(pos_s, [r, q], pos, mask=m)
        plsc.store_scatter(org_s, [r, q], org, mask=m)
        plsc.store_scatter(adj_s, [r, q], adj, mask=m)
        plsc.store_scatter(adjc_s, [r, q], adjc, mask=m)

    for p in range(PPC):
        r0 = (c * PPC + p) * LMAX
        r0v = jnp.broadcast_to(r0, (LANES,))
        hiv = r0v + LMAX

        def grp(g, cnt):
            v = idx_v[pl.ds(g * LANES, LANES)]
            m = (v >= r0v) & (v < hiv)
            pos = jnp.broadcast_to(row0 + g * LANES, (LANES,)) + iota
            adj = jnp.where(m, v - r0v, DUMP + (pos & (NDUMP - 1)))
            dst = jnp.broadcast_to(cnt, (LANES,)) + plsc.cumsum(
                m.astype(jnp.int32)) - 1
            put(dst, pos, v, adj, adj, m)
            return cnt + jnp.max(plsc.all_reduce_population_count(m))

        cnt = lax.fori_loop(0, NG, grp, jnp.int32(0))

        p0 = _first(pos_s[0, pl.ds(0, LANES)])
        o0 = _first(org_s[0, pl.ds(0, LANES)])
        a0 = _first(adj_s[0, pl.ds(0, LANES)])
        for t in range(IB // LANES):
            dst = jnp.broadcast_to(cnt + t * LANES, (LANES,)) + iota
            put(dst, p0, o0, a0, DUMP + (dst & (NDUMP - 1)), None)

        for j in range(NB):
            @pl.when(j * IB < cnt)
            def _():
                pltpu.sync_copy(mem_hbm.at[org_s.at[j]], rowbuf)
                pltpu.sync_copy(rowbuf, acc_sh.at[adj_s.at[j]])
        plsc.subcore_barrier()

        for j in range(NB):
            @pl.when(j * IB < cnt)
            def _():
                pltpu.sync_copy(upd_hbm.at[pos_s.at[j]], rowbuf)
                pltpu.sync_copy(rowbuf, acc_sh.at[adjc_s.at[j]], add=True)
        plsc.subcore_barrier()

        for j in range(NB):
            @pl.when(j * IB < cnt)
            def _():
                pltpu.sync_copy(acc_sh.at[adj_s.at[j]], rowbuf)
                pltpu.sync_copy(rowbuf, out_hbm.at[pos_s.at[j]])
        plsc.subcore_barrier()


_sc_scatter_gather = functools.partial(
    pl.kernel,
    out_type=jax.ShapeDtypeStruct((B, D), jnp.float32),
    mesh=plsc.VectorSubcoreMesh(core_axis_name="c", subcore_axis_name="s"),
    scratch_types=[
        pltpu.VMEM((BPT,), jnp.int32),
        pltpu.VMEM((LROWS, IB), jnp.int32),
        pltpu.VMEM((LROWS, IB), jnp.int32),
        pltpu.VMEM((LROWS, IB), jnp.int32),
        pltpu.VMEM((LROWS, IB), jnp.int32),
        pltpu.VMEM((IB, D), jnp.float32),
        pltpu.VMEM_SHARED((DUMP + NDUMP, D), jnp.float32),
    ],
    compiler_params=pltpu.CompilerParams(needs_layout_passes=False),
)(_sc_body)


def kernel(mem, idx, val, W1, b1, W2, b2):
    update = _mlp(val, W1, b1.reshape(1, H), W2, b2.reshape(1, D))
    return _sc_scatter_gather(idx.astype(jnp.int32), update, mem)


# speedup vs baseline: 1.3647x; 1.1292x over previous
import functools

import jax
import jax.numpy as jnp
from jax import lax
from jax.experimental import pallas as pl
from jax.experimental.pallas import tpu as pltpu
from jax.experimental.pallas import tpu_sc as plsc

M = 100000
D = 128
H = 256
B = 16384

NC = 2
NS = 16
LANES = 16

PPC = 4
LMAX = 12544
RPT = LMAX // NS
DUMP = LMAX
NDUMP = 64
BPT = B // NS
NG = BPT // LANES
IB = 128
NB = BPT // IB
LROWS = NB + 2

BM = 2048


def _mlp_body(val_ref, w1_ref, b1_ref, w2_ref, b2_ref, out_ref):
    h = jnp.dot(val_ref[...], w1_ref[...], preferred_element_type=jnp.float32)
    h = jnp.maximum(h + b1_ref[...], 0.0)
    out_ref[...] = (
        jnp.dot(h, w2_ref[...], preferred_element_type=jnp.float32) + b2_ref[...]
    )


_mlp = pl.pallas_call(
    _mlp_body,
    grid=(B // BM,),
    in_specs=[
        pl.BlockSpec((BM, D), lambda i: (i, 0)),
        pl.BlockSpec((D, H), lambda i: (0, 0)),
        pl.BlockSpec((1, H), lambda i: (0, 0)),
        pl.BlockSpec((H, D), lambda i: (0, 0)),
        pl.BlockSpec((1, D), lambda i: (0, 0)),
    ],
    out_specs=pl.BlockSpec((BM, D), lambda i: (i, 0)),
    out_shape=jax.ShapeDtypeStruct((B, D), jnp.float32),
)


def _first(vec):
    e0 = jnp.sum(jnp.where(lax.iota(jnp.int32, LANES) == 0, vec, 0))
    return jnp.broadcast_to(e0, (LANES,))


def _sc_body(idx_hbm, upd_hbm, mem_hbm, out_hbm,
             idx_v, pos_s, adj_s, adjc_s, rowbuf, acc_sh):
    c = lax.axis_index("c")
    s = lax.axis_index("s")
    row0 = s * BPT
    pltpu.sync_copy(idx_hbm.at[pl.ds(row0, BPT)], idx_v)
    iota = lax.iota(jnp.int32, LANES)

    def put(dst, pos, adj, adjc, m):
        r = lax.shift_right_logical(dst, 7)
        q = dst & (IB - 1)
        plsc.store_scatter(pos_s, [r, q], pos, mask=m)
        plsc.store_scatter(adj_s, [r, q], adj, mask=m)
        plsc.store_scatter(adjc_s, [r, q], adjc, mask=m)

    for p in range(PPC):
        r0 = (c * PPC + p) * LMAX
        r0v = jnp.broadcast_to(r0, (LANES,))
        hiv = r0v + LMAX

        def grp(g, cnt):
            v = idx_v[pl.ds(g * LANES, LANES)]
            m = (v >= r0v) & (v < hiv)
            pos = jnp.broadcast_to(row0 + g * LANES, (LANES,)) + iota
            adj = jnp.where(m, v - r0v, DUMP + (pos & (NDUMP - 1)))
            dst = jnp.broadcast_to(cnt, (LANES,)) + plsc.cumsum(
                m.astype(jnp.int32)) - 1
            put(dst, pos, adj, adj, m)
            return cnt + jnp.max(plsc.all_reduce_population_count(m))

        cnt = lax.fori_loop(0, NG, grp, jnp.int32(0))

        p0 = _first(pos_s[0, pl.ds(0, LANES)])
        a0 = _first(adj_s[0, pl.ds(0, LANES)])
        for t in range(IB // LANES):
            dst = jnp.broadcast_to(cnt + t * LANES, (LANES,)) + iota
            put(dst, p0, a0, DUMP + (dst & (NDUMP - 1)), None)

        start = r0 + s * RPT
        @pl.when(start + RPT <= M)
        def _():
            pltpu.sync_copy(mem_hbm.at[pl.ds(start, RPT), :],
                            acc_sh.at[pl.ds(s * RPT, RPT), :])
        @pl.when((start < M) & (start + RPT > M))
        def _():
            pltpu.sync_copy(mem_hbm.at[pl.ds(start, M % RPT), :],
                            acc_sh.at[pl.ds(s * RPT, M % RPT), :])
        plsc.subcore_barrier()

        for j in range(NB):
            @pl.when(j * IB < cnt)
            def _():
                pltpu.sync_copy(upd_hbm.at[pos_s.at[j]], rowbuf)
                pltpu.sync_copy(rowbuf, acc_sh.at[adjc_s.at[j]], add=True)
        plsc.subcore_barrier()

        for j in range(NB):
            @pl.when(j * IB < cnt)
            def _():
                pltpu.sync_copy(acc_sh.at[adj_s.at[j]], rowbuf)
                pltpu.sync_copy(rowbuf, out_hbm.at[pos_s.at[j]])
        plsc.subcore_barrier()


_sc_scatter_gather = functools.partial(
    pl.kernel,
    out_type=jax.ShapeDtypeStruct((B, D), jnp.float32),
    mesh=plsc.VectorSubcoreMesh(core_axis_name="c", subcore_axis_name="s"),
    scratch_types=[
        pltpu.VMEM((BPT,), jnp.int32),
        pltpu.VMEM((LROWS, IB), jnp.int32),
        pltpu.VMEM((LROWS, IB), jnp.int32),
        pltpu.VMEM((LROWS, IB), jnp.int32),
        pltpu.VMEM((IB, D), jnp.float32),
        pltpu.VMEM_SHARED((DUMP + NDUMP, D), jnp.float32),
    ],
    compiler_params=pltpu.CompilerParams(needs_layout_passes=False),
)(_sc_body)


def kernel(mem, idx, val, W1, b1, W2, b2):
    update = _mlp(val, W1, b1.reshape(1, H), W2, b2.reshape(1, D))
    return _sc_scatter_gather(idx.astype(jnp.int32), update, mem)
